# Initial kernel scaffold; baseline (speedup 1.0000x reference)
import jax
import jax.numpy as jnp
from jax.experimental import pallas as pl


def kernel(z, edge_index, l0_W1, l0_b1, l0_W2, l0_b2, l1_W1, l1_b1, l1_W2, l1_b2):
    raise NotImplementedError("write your pallas kernel here")


# speedup vs baseline: 1.4889x; 1.4889x over previous
import functools

import jax
import jax.numpy as jnp
from jax import lax
from jax.experimental import pallas as pl
from jax.experimental.pallas import tpu as pltpu
from jax.experimental.pallas import tpu_sc as plsc

N_NODES = 10000
N_EDGES = 320000
D = 128

NC = 2
NS = 16
NW = NC * NS
L = 16

NPT = 320
N_PAD = NW * NPT
EPT = N_EDGES // NW

SCAN_CH = 6400
FLUSH = 8192
BUFW = FLUSH + SCAN_CH + L
CAP = 40 * FLUSH
KG = 80
KS = 128

NEG_INIT = float("-inf")

_SC_PARAMS = pltpu.CompilerParams(needs_layout_passes=False)


def _wid():
  return lax.axis_index("s") * NC + lax.axis_index("c")


def _iota():
  return lax.iota(jnp.int32, L)


def _prepass_body(dst_hbm, ids_hbm, dstl_hbm, cnts_hbm, stage, idbuf, dlbuf,
                  cntv):
  wid = _wid()
  lo = wid * NPT
  hi = lo + NPT

  def memset_body(i, _):
    idbuf[pl.ds(i * L, L)] = jnp.zeros((L,), jnp.int32)
    dlbuf[pl.ds(i * L, L)] = jnp.full((L,), NPT, jnp.int32)
    return 0
  lax.fori_loop(0, BUFW // L, memset_body, 0)

  n_chunks = N_EDGES // SCAN_CH

  def chunk_body(ch, carry):
    cnt, nfl = carry
    pltpu.sync_copy(dst_hbm.at[pl.ds(ch * SCAN_CH, SCAN_CH)], stage)

    def group_body(g, cnt):
      dv = stage[pl.ds(g * L, L)]
      m = (dv >= lo) & (dv < hi)
      idv = _iota() + (ch * SCAN_CH + g * L)
      csv = plsc.cumsum(m.astype(jnp.int32))
      pos = csv + (cnt - 1)
      plsc.store_scatter(idbuf, [pos], idv, mask=m)
      plsc.store_scatter(dlbuf, [pos], dv - lo, mask=m)
      return cnt + jnp.max(csv)

    cnt = lax.fori_loop(0, SCAN_CH // L, group_body, cnt)

    def do_flush(c):
      cnt, nfl = c
      pltpu.sync_copy(idbuf.at[pl.ds(0, FLUSH)],
                      ids_hbm.at[wid, pl.ds(nfl * FLUSH, FLUSH)])
      pltpu.sync_copy(dlbuf.at[pl.ds(0, FLUSH)],
                      dstl_hbm.at[wid, pl.ds(nfl * FLUSH, FLUSH)])
      rem = cnt - FLUSH

      def shift_body(k, _):
        idbuf[pl.ds(k * L, L)] = idbuf[pl.ds(FLUSH + k * L, L)]
        dlbuf[pl.ds(k * L, L)] = dlbuf[pl.ds(FLUSH + k * L, L)]
        return 0
      lax.fori_loop(0, (rem + L - 1) // L, shift_body, 0)
      return rem, nfl + 1

    return lax.cond(cnt >= FLUSH, do_flush, lambda c: c, (cnt, nfl))

  cnt, nfl = lax.fori_loop(0, n_chunks, chunk_body,
                           (jnp.int32(0), jnp.int32(0)))

  pltpu.sync_copy(idbuf.at[pl.ds(0, FLUSH)],
                  ids_hbm.at[wid, pl.ds(nfl * FLUSH, FLUSH)])
  pltpu.sync_copy(dlbuf.at[pl.ds(0, FLUSH)],
                  dstl_hbm.at[wid, pl.ds(nfl * FLUSH, FLUSH)])

  total = nfl * FLUSH + cnt
  cntv[...] = jnp.zeros((L,), jnp.int32) + total
  pltpu.sync_copy(cntv, cnts_hbm.at[pl.ds(wid * L, L)])


def _sc_prepass(dst):
  mesh = plsc.VectorSubcoreMesh(core_axis_name="c", subcore_axis_name="s")
  f = pl.kernel(
      _prepass_body,
      out_type=(
          jax.ShapeDtypeStruct((NW, CAP), jnp.int32),
          jax.ShapeDtypeStruct((NW, CAP), jnp.int32),
          jax.ShapeDtypeStruct((NW * L,), jnp.int32),
      ),
      mesh=mesh,
      compiler_params=_SC_PARAMS,
      scratch_types=[
          pltpu.VMEM((SCAN_CH,), jnp.int32),
          pltpu.VMEM((BUFW,), jnp.int32),
          pltpu.VMEM((BUFW,), jnp.int32),
          pltpu.VMEM((L,), jnp.int32),
      ],
  )
  return f(dst)


def _gather_body(a_hbm, b_hbm, src_hbm, dst_hbm, out_hbm, sidx, didx, rows_a,
                 rows_b, sem_a, sem_b):
  wid = _wid()
  e0 = wid * EPT
  cols = [c * L + _iota() for c in range(D // L)]

  def batch_body(g, _):
    base = e0 + g * KG
    pltpu.sync_copy(src_hbm.at[pl.ds(base, KG)], sidx)
    pltpu.sync_copy(dst_hbm.at[pl.ds(base, KG)], didx)
    cp_a = pltpu.async_copy(a_hbm.at[didx], rows_a, sem_a)
    cp_b = pltpu.async_copy(b_hbm.at[sidx], rows_b, sem_b)
    cp_a.wait()
    cp_b.wait()

    def row_body(i, _):
      iv = jnp.zeros((L,), jnp.int32) + i
      for c in range(D // L):
        av = plsc.load_gather(rows_a, [iv, cols[c]])
        bv = plsc.load_gather(rows_b, [iv, cols[c]])
        plsc.store_scatter(rows_a, [iv, cols[c]],
                           jnp.maximum(av + bv, 0.0))
      return 0
    lax.fori_loop(0, KG, row_body, 0)

    pltpu.sync_copy(rows_a, out_hbm.at[pl.ds(base, KG)])
    return 0

  lax.fori_loop(0, EPT // KG, batch_body, 0)


def _sc_gather(a, b, src, dst):
  mesh = plsc.VectorSubcoreMesh(core_axis_name="c", subcore_axis_name="s")
  f = pl.kernel(
      _gather_body,
      out_type=jax.ShapeDtypeStruct((N_EDGES, D), jnp.float32),
      mesh=mesh,
      compiler_params=_SC_PARAMS,
      scratch_types=[
          pltpu.VMEM((KG,), jnp.int32),
          pltpu.VMEM((KG,), jnp.int32),
          pltpu.VMEM((KG, D), jnp.float32),
          pltpu.VMEM((KG, D), jnp.float32),
          pltpu.SemaphoreType.DMA,
          pltpu.SemaphoreType.DMA,
      ],
  )
  return f(a, b, src, dst)


def _scatter_body(m_hbm, ids_hbm, dstl_hbm, cnts_hbm, out_hbm, idxb, dlb,
                  rows, cntv, acc, sem, *, init_val, finalize):
  wid = _wid()
  lo = wid * NPT
  cols = [c * L + _iota() for c in range(D // L)]

  def init_body(i, _):
    acc[pl.ds(i * L, L)] = jnp.full((L,), init_val, jnp.float32)
    return 0
  lax.fori_loop(0, (NPT + 1) * D // L, init_body, 0)

  pltpu.sync_copy(cnts_hbm.at[pl.ds(wid * L, L)], cntv)
  cnt = jnp.max(cntv[...])
  nb = (cnt + KS - 1) // KS

  def batch_body(g, _):
    off = g * KS
    pltpu.sync_copy(ids_hbm.at[wid, pl.ds(off, KS)], idxb)
    pltpu.sync_copy(dstl_hbm.at[wid, pl.ds(off, KS)], dlb)
    pltpu.async_copy(m_hbm.at[idxb], rows, sem).wait()

    def edge_body(j, _):
      jv = jnp.zeros((L,), jnp.int32) + j
      dj = plsc.load_gather(dlb, [jv])
      for c in range(D // L):
        idx = dj * D + cols[c]
        cur = plsc.load_gather(acc, [idx])
        val = plsc.load_gather(rows, [jv, cols[c]])
        plsc.store_scatter(acc, [idx], jnp.maximum(cur, val))
      return 0
    lax.fori_loop(0, KS, edge_body, 0)
    return 0

  lax.fori_loop(0, nb, batch_body, 0)

  if finalize:
    def fin_body(i, _):
      sl = pl.ds(i * L, L)
      v = acc[sl]
      acc[sl] = jnp.where(v == jnp.float32(NEG_INIT), 0.0, v)
      return 0
    lax.fori_loop(0, NPT * D // L, fin_body, 0)

  pltpu.sync_copy(acc.at[pl.ds(0, NPT * D)],
                  out_hbm.at[pl.ds(lo * D, NPT * D)])


def _sc_scatter(m, ids, dstl, cnts, init_val, finalize):
  mesh = plsc.VectorSubcoreMesh(core_axis_name="c", subcore_axis_name="s")
  body = functools.partial(_scatter_body, init_val=init_val, finalize=finalize)
  f = pl.kernel(
      body,
      out_type=jax.ShapeDtypeStruct((N_PAD * D,), jnp.float32),
      mesh=mesh,
      compiler_params=_SC_PARAMS,
      scratch_types=[
          pltpu.VMEM((KS,), jnp.int32),
          pltpu.VMEM((KS,), jnp.int32),
          pltpu.VMEM((KS, D), jnp.float32),
          pltpu.VMEM((L,), jnp.int32),
          pltpu.VMEM(((NPT + 1) * D,), jnp.float32),
          pltpu.SemaphoreType.DMA,
      ],
  )
  return f(m, ids, dstl, cnts)


def _ab_body(x_ref, wa_ref, wb_ref, b_ref, oa_ref, ob_ref):
  x = x_ref[...]
  oa_ref[...] = (
      jnp.dot(x, wa_ref[...], preferred_element_type=jnp.float32) + b_ref[...]
  )
  ob_ref[...] = jnp.dot(x, wb_ref[...], preferred_element_type=jnp.float32)


def _tc_ab(x, wa, wb, b1, bm):
  m = x.shape[0]
  return pl.pallas_call(
      _ab_body,
      grid=(m // bm,),
      in_specs=[
          pl.BlockSpec((bm, D), lambda i: (i, 0)),
          pl.BlockSpec((D, D), lambda i: (0, 0)),
          pl.BlockSpec((D, D), lambda i: (0, 0)),
          pl.BlockSpec((1, D), lambda i: (0, 0)),
      ],
      out_specs=[
          pl.BlockSpec((bm, D), lambda i: (i, 0)),
          pl.BlockSpec((bm, D), lambda i: (i, 0)),
      ],
      out_shape=[
          jax.ShapeDtypeStruct((m, D), jnp.float32),
          jax.ShapeDtypeStruct((m, D), jnp.float32),
      ],
  )(x, wa, wb, b1.reshape(1, D))


def _mm_body(x_ref, w_ref, b_ref, o_ref):
  o_ref[...] = (
      jnp.dot(x_ref[...], w_ref[...], preferred_element_type=jnp.float32)
      + b_ref[...]
  )


def _tc_mm(x, w, b, bm):
  m, k = x.shape
  n = w.shape[1]
  return pl.pallas_call(
      _mm_body,
      grid=(m // bm,),
      in_specs=[
          pl.BlockSpec((bm, k), lambda i: (i, 0)),
          pl.BlockSpec((k, n), lambda i: (0, 0)),
          pl.BlockSpec((1, n), lambda i: (0, 0)),
      ],
      out_specs=pl.BlockSpec((bm, n), lambda i: (i, 0)),
      out_shape=jax.ShapeDtypeStruct((m, n), jnp.float32),
  )(x, w, b.reshape(1, n))


def kernel(z, edge_index, l0_W1, l0_b1, l0_W2, l0_b2, l1_W1, l1_b1, l1_W2,
           l1_b2):
  src = edge_index[0].astype(jnp.int32)
  dst = edge_index[1].astype(jnp.int32)

  ids, dstl, cnts = _sc_prepass(dst)

  h = jnp.zeros((N_PAD, D), jnp.float32).at[:N_NODES].set(z)
  for li, (W1, b1, W2, b2) in enumerate(
      [(l0_W1, l0_b1, l0_W2, l0_b2), (l1_W1, l1_b1, l1_W2, l1_b2)]):
    last = li == 1
    a, bmat = _tc_ab(h, W1[:D], W1[D:], b1, bm=1024)
    m1 = _sc_gather(a, bmat, src, dst)
    mm = _tc_mm(m1, W2, b2, bm=2000)
    init = 0.0 if not last else NEG_INIT
    hflat = _sc_scatter(mm, ids, dstl, cnts, init, finalize=last)
    h = hflat.reshape(N_PAD, D)

  return h[:N_NODES]


# speedup vs baseline: 1.8214x; 1.2233x over previous
import functools

import jax
import jax.numpy as jnp
from jax import lax
from jax.experimental import pallas as pl
from jax.experimental.pallas import tpu as pltpu
from jax.experimental.pallas import tpu_sc as plsc

N_NODES = 10000
N_EDGES = 320000
D = 128

NC = 2
NS = 16
NW = NC * NS
L = 16

NPT = 320
N_PAD = NW * NPT
EPT = N_EDGES // NW

SCAN_CH = 6400
FLUSH = 8192
BUFW = FLUSH + SCAN_CH + L
CAP = 40 * FLUSH
KG = 80
KS = 128

NEG_INIT = float("-inf")

_SC_PARAMS = pltpu.CompilerParams(needs_layout_passes=False)


def _wid():
  return lax.axis_index("s") * NC + lax.axis_index("c")


def _iota():
  return lax.iota(jnp.int32, L)


def _prepass_body(dst_hbm, ids_hbm, dstl_hbm, cnts_hbm, stage, idbuf, dlbuf,
                  cntv):
  wid = _wid()
  lo = wid * NPT
  hi = lo + NPT

  def memset_body(i, _):
    idbuf[pl.ds(i * L, L)] = jnp.zeros((L,), jnp.int32)
    dlbuf[pl.ds(i * L, L)] = jnp.full((L,), NPT, jnp.int32)
    return 0
  lax.fori_loop(0, BUFW // L, memset_body, 0)

  n_chunks = N_EDGES // SCAN_CH

  def chunk_body(ch, carry):
    cnt, nfl = carry
    pltpu.sync_copy(dst_hbm.at[pl.ds(ch * SCAN_CH, SCAN_CH)], stage)

    def group_body(g, cnt):
      dv = stage[pl.ds(g * L, L)]
      m = (dv >= lo) & (dv < hi)
      idv = _iota() + (ch * SCAN_CH + g * L)
      csv = plsc.cumsum(m.astype(jnp.int32))
      pos = csv + (cnt - 1)
      plsc.store_scatter(idbuf, [pos], idv, mask=m)
      plsc.store_scatter(dlbuf, [pos], dv - lo, mask=m)
      return cnt + jnp.max(csv)

    cnt = lax.fori_loop(0, SCAN_CH // L, group_body, cnt)

    def do_flush(c):
      cnt, nfl = c
      pltpu.sync_copy(idbuf.at[pl.ds(0, FLUSH)],
                      ids_hbm.at[wid, pl.ds(nfl * FLUSH, FLUSH)])
      pltpu.sync_copy(dlbuf.at[pl.ds(0, FLUSH)],
                      dstl_hbm.at[wid, pl.ds(nfl * FLUSH, FLUSH)])
      rem = cnt - FLUSH

      def shift_body(k, _):
        idbuf[pl.ds(k * L, L)] = idbuf[pl.ds(FLUSH + k * L, L)]
        dlbuf[pl.ds(k * L, L)] = dlbuf[pl.ds(FLUSH + k * L, L)]
        return 0
      lax.fori_loop(0, (rem + L - 1) // L, shift_body, 0)
      return rem, nfl + 1

    return lax.cond(cnt >= FLUSH, do_flush, lambda c: c, (cnt, nfl))

  cnt, nfl = lax.fori_loop(0, n_chunks, chunk_body,
                           (jnp.int32(0), jnp.int32(0)))

  pltpu.sync_copy(idbuf.at[pl.ds(0, FLUSH)],
                  ids_hbm.at[wid, pl.ds(nfl * FLUSH, FLUSH)])
  pltpu.sync_copy(dlbuf.at[pl.ds(0, FLUSH)],
                  dstl_hbm.at[wid, pl.ds(nfl * FLUSH, FLUSH)])

  total = nfl * FLUSH + cnt
  cntv[...] = jnp.zeros((L,), jnp.int32) + total
  pltpu.sync_copy(cntv, cnts_hbm.at[pl.ds(wid * L, L)])


def _sc_prepass(dst):
  mesh = plsc.VectorSubcoreMesh(core_axis_name="c", subcore_axis_name="s")
  f = pl.kernel(
      _prepass_body,
      out_type=(
          jax.ShapeDtypeStruct((NW, CAP), jnp.int32),
          jax.ShapeDtypeStruct((NW, CAP), jnp.int32),
          jax.ShapeDtypeStruct((NW * L,), jnp.int32),
      ),
      mesh=mesh,
      compiler_params=_SC_PARAMS,
      scratch_types=[
          pltpu.VMEM((SCAN_CH,), jnp.int32),
          pltpu.VMEM((BUFW,), jnp.int32),
          pltpu.VMEM((BUFW,), jnp.int32),
          pltpu.VMEM((L,), jnp.int32),
      ],
  )
  return f(dst)


NB_G = EPT // KG
NBP_G = NB_G + 3


def _gather_body(a_hbm, b_hbm, src_hbm, dst_hbm, out_hbm, sidx, didx,
                 ina0, ina1, inb0, inb1, outb0, outb1,
                 sa0, sa1, sb0, sb1, so0, so1):
  wid = _wid()
  e0 = wid * EPT
  cols = [c * L + _iota() for c in range(D // L)]
  ina = (ina0, ina1)
  inb = (inb0, inb1)
  outb = (outb0, outb1)
  sa = (sa0, sa1)
  sb = (sb0, sb1)
  so = (so0, so1)

  pltpu.sync_copy(src_hbm.at[pl.ds(e0, EPT)], sidx)
  pltpu.sync_copy(dst_hbm.at[pl.ds(e0, EPT)], didx)

  def start_in(g, b):
    off = g * KG
    pltpu.async_copy(a_hbm.at[didx.at[pl.ds(off, KG)]], ina[b], sa[b])
    pltpu.async_copy(b_hbm.at[sidx.at[pl.ds(off, KG)]], inb[b], sb[b])

  start_in(0, 0)
  start_in(1, 1)

  def outer(t, _):
    for b in range(2):
      g = t * 2 + b

      @pl.when(g < NB_G)
      def _():
        pltpu.make_async_copy(a_hbm.at[pl.ds(0, KG)], ina[b], sa[b]).wait()
        pltpu.make_async_copy(b_hbm.at[pl.ds(0, KG)], inb[b], sb[b]).wait()

      @pl.when((g >= 2) & (g < NB_G + 2))
      def _():
        pltpu.make_async_copy(outb[b], out_hbm.at[pl.ds(0, KG)], so[b]).wait()

      @pl.when(g < NB_G)
      def _():
        def row_body(i, _):
          iv = jnp.zeros((L,), jnp.int32) + i
          for c in range(D // L):
            av = plsc.load_gather(ina[b], [iv, cols[c]])
            bv = plsc.load_gather(inb[b], [iv, cols[c]])
            plsc.store_scatter(outb[b], [iv, cols[c]],
                               jnp.maximum(av + bv, 0.0))
          return 0
        lax.fori_loop(0, KG, row_body, 0)
        pltpu.async_copy(outb[b], out_hbm.at[pl.ds(e0 + g * KG, KG)], so[b])

      @pl.when(g + 2 < NB_G)
      def _():
        start_in(g + 2, b)
    return 0

  lax.fori_loop(0, (NBP_G + 1) // 2, outer, 0)


def _sc_gather(a, b, src, dst):
  mesh = plsc.VectorSubcoreMesh(core_axis_name="c", subcore_axis_name="s")
  f = pl.kernel(
      _gather_body,
      out_type=jax.ShapeDtypeStruct((N_EDGES, D), jnp.float32),
      mesh=mesh,
      compiler_params=_SC_PARAMS,
      scratch_types=[
          pltpu.VMEM((EPT,), jnp.int32),
          pltpu.VMEM((EPT,), jnp.int32),
          pltpu.VMEM((KG, D), jnp.float32),
          pltpu.VMEM((KG, D), jnp.float32),
          pltpu.VMEM((KG, D), jnp.float32),
          pltpu.VMEM((KG, D), jnp.float32),
          pltpu.VMEM((KG, D), jnp.float32),
          pltpu.VMEM((KG, D), jnp.float32),
          pltpu.SemaphoreType.DMA,
          pltpu.SemaphoreType.DMA,
          pltpu.SemaphoreType.DMA,
          pltpu.SemaphoreType.DMA,
          pltpu.SemaphoreType.DMA,
          pltpu.SemaphoreType.DMA,
      ],
  )
  return f(a, b, src, dst)


def _scatter_body(m_hbm, ids_hbm, dstl_hbm, cnts_hbm, out_hbm, idxb, dlb,
                  rows, cntv, acc, sem, *, init_val, finalize):
  wid = _wid()
  lo = wid * NPT
  cols = [c * L + _iota() for c in range(D // L)]

  def init_body(i, _):
    acc[pl.ds(i * L, L)] = jnp.full((L,), init_val, jnp.float32)
    return 0
  lax.fori_loop(0, (NPT + 1) * D // L, init_body, 0)

  pltpu.sync_copy(cnts_hbm.at[pl.ds(wid * L, L)], cntv)
  cnt = jnp.max(cntv[...])
  nb = (cnt + KS - 1) // KS

  def batch_body(g, _):
    off = g * KS
    pltpu.sync_copy(ids_hbm.at[wid, pl.ds(off, KS)], idxb)
    pltpu.sync_copy(dstl_hbm.at[wid, pl.ds(off, KS)], dlb)
    pltpu.async_copy(m_hbm.at[idxb], rows, sem).wait()

    def edge_body(j, _):
      jv = jnp.zeros((L,), jnp.int32) + j
      dj = plsc.load_gather(dlb, [jv])
      for c in range(D // L):
        idx = dj * D + cols[c]
        cur = plsc.load_gather(acc, [idx])
        val = plsc.load_gather(rows, [jv, cols[c]])
        plsc.store_scatter(acc, [idx], jnp.maximum(cur, val))
      return 0
    lax.fori_loop(0, KS, edge_body, 0)
    return 0

  lax.fori_loop(0, nb, batch_body, 0)

  if finalize:
    def fin_body(i, _):
      sl = pl.ds(i * L, L)
      v = acc[sl]
      acc[sl] = jnp.where(v == jnp.float32(NEG_INIT), 0.0, v)
      return 0
    lax.fori_loop(0, NPT * D // L, fin_body, 0)

  pltpu.sync_copy(acc.at[pl.ds(0, NPT * D)],
                  out_hbm.at[pl.ds(lo * D, NPT * D)])


def _sc_scatter(m, ids, dstl, cnts, init_val, finalize):
  mesh = plsc.VectorSubcoreMesh(core_axis_name="c", subcore_axis_name="s")
  body = functools.partial(_scatter_body, init_val=init_val, finalize=finalize)
  f = pl.kernel(
      body,
      out_type=jax.ShapeDtypeStruct((N_PAD * D,), jnp.float32),
      mesh=mesh,
      compiler_params=_SC_PARAMS,
      scratch_types=[
          pltpu.VMEM((KS,), jnp.int32),
          pltpu.VMEM((KS,), jnp.int32),
          pltpu.VMEM((KS, D), jnp.float32),
          pltpu.VMEM((L,), jnp.int32),
          pltpu.VMEM(((NPT + 1) * D,), jnp.float32),
          pltpu.SemaphoreType.DMA,
      ],
  )
  return f(m, ids, dstl, cnts)


def _ab_body(x_ref, wa_ref, wb_ref, b_ref, oa_ref, ob_ref):
  x = x_ref[...]
  oa_ref[...] = (
      jnp.dot(x, wa_ref[...], preferred_element_type=jnp.float32) + b_ref[...]
  )
  ob_ref[...] = jnp.dot(x, wb_ref[...], preferred_element_type=jnp.float32)


def _tc_ab(x, wa, wb, b1, bm):
  m = x.shape[0]
  return pl.pallas_call(
      _ab_body,
      grid=(m // bm,),
      in_specs=[
          pl.BlockSpec((bm, D), lambda i: (i, 0)),
          pl.BlockSpec((D, D), lambda i: (0, 0)),
          pl.BlockSpec((D, D), lambda i: (0, 0)),
          pl.BlockSpec((1, D), lambda i: (0, 0)),
      ],
      out_specs=[
          pl.BlockSpec((bm, D), lambda i: (i, 0)),
          pl.BlockSpec((bm, D), lambda i: (i, 0)),
      ],
      out_shape=[
          jax.ShapeDtypeStruct((m, D), jnp.float32),
          jax.ShapeDtypeStruct((m, D), jnp.float32),
      ],
  )(x, wa, wb, b1.reshape(1, D))


def _mm_body(x_ref, w_ref, b_ref, o_ref):
  o_ref[...] = (
      jnp.dot(x_ref[...], w_ref[...], preferred_element_type=jnp.float32)
      + b_ref[...]
  )


def _tc_mm(x, w, b, bm):
  m, k = x.shape
  n = w.shape[1]
  return pl.pallas_call(
      _mm_body,
      grid=(m // bm,),
      in_specs=[
          pl.BlockSpec((bm, k), lambda i: (i, 0)),
          pl.BlockSpec((k, n), lambda i: (0, 0)),
          pl.BlockSpec((1, n), lambda i: (0, 0)),
      ],
      out_specs=pl.BlockSpec((bm, n), lambda i: (i, 0)),
      out_shape=jax.ShapeDtypeStruct((m, n), jnp.float32),
  )(x, w, b.reshape(1, n))


def kernel(z, edge_index, l0_W1, l0_b1, l0_W2, l0_b2, l1_W1, l1_b1, l1_W2,
           l1_b2):
  src = edge_index[0].astype(jnp.int32)
  dst = edge_index[1].astype(jnp.int32)

  ids, dstl, cnts = _sc_prepass(dst)

  h = jnp.zeros((N_PAD, D), jnp.float32).at[:N_NODES].set(z)
  for li, (W1, b1, W2, b2) in enumerate(
      [(l0_W1, l0_b1, l0_W2, l0_b2), (l1_W1, l1_b1, l1_W2, l1_b2)]):
    last = li == 1
    a, bmat = _tc_ab(h, W1[:D], W1[D:], b1, bm=1024)
    m1 = _sc_gather(a, bmat, src, dst)
    mm = _tc_mm(m1, W2, b2, bm=2000)
    init = 0.0 if not last else NEG_INIT
    hflat = _sc_scatter(mm, ids, dstl, cnts, init, finalize=last)
    h = hflat.reshape(N_PAD, D)

  return h[:N_NODES]


# speedup vs baseline: 2.2313x; 1.2251x over previous
import functools

import jax
import jax.numpy as jnp
from jax import lax
from jax.experimental import pallas as pl
from jax.experimental.pallas import tpu as pltpu
from jax.experimental.pallas import tpu_sc as plsc

N_NODES = 10000
N_EDGES = 320000
D = 128

NC = 2
NS = 16
NW = NC * NS
L = 16

NPT = 320
N_PAD = NW * NPT
EPT = N_EDGES // NW

SCAN_CH = 6400
FLUSH = 8192
BUFW = FLUSH + SCAN_CH + L
CAP = 40 * FLUSH
KG = 80
KS = 128

NEG_INIT = float("-inf")

_SC_PARAMS = pltpu.CompilerParams(needs_layout_passes=False)


def _wid():
  return lax.axis_index("s") * NC + lax.axis_index("c")


def _iota():
  return lax.iota(jnp.int32, L)


def _prepass_body(dst_hbm, ids_hbm, dstl_hbm, cnts_hbm, stage, idbuf, dlbuf,
                  cntv):
  wid = _wid()
  lo = wid * NPT
  hi = lo + NPT

  def memset_body(i, _):
    idbuf[pl.ds(i * L, L)] = jnp.zeros((L,), jnp.int32)
    dlbuf[pl.ds(i * L, L)] = jnp.full((L,), NPT, jnp.int32)
    return 0
  lax.fori_loop(0, BUFW // L, memset_body, 0)

  n_chunks = N_EDGES // SCAN_CH

  def chunk_body(ch, carry):
    cnt, nfl = carry
    pltpu.sync_copy(dst_hbm.at[pl.ds(ch * SCAN_CH, SCAN_CH)], stage)

    def group_body(g, cnt):
      dv = stage[pl.ds(g * L, L)]
      m = (dv >= lo) & (dv < hi)
      idv = _iota() + (ch * SCAN_CH + g * L)
      csv = plsc.cumsum(m.astype(jnp.int32))
      pos = csv + (cnt - 1)
      plsc.store_scatter(idbuf, [pos], idv, mask=m)
      plsc.store_scatter(dlbuf, [pos], dv - lo, mask=m)
      return cnt + jnp.max(csv)

    cnt = lax.fori_loop(0, SCAN_CH // L, group_body, cnt)

    def do_flush(c):
      cnt, nfl = c
      pltpu.sync_copy(idbuf.at[pl.ds(0, FLUSH)],
                      ids_hbm.at[wid, pl.ds(nfl * FLUSH, FLUSH)])
      pltpu.sync_copy(dlbuf.at[pl.ds(0, FLUSH)],
                      dstl_hbm.at[wid, pl.ds(nfl * FLUSH, FLUSH)])
      rem = cnt - FLUSH

      def shift_body(k, _):
        idbuf[pl.ds(k * L, L)] = idbuf[pl.ds(FLUSH + k * L, L)]
        dlbuf[pl.ds(k * L, L)] = dlbuf[pl.ds(FLUSH + k * L, L)]
        return 0
      lax.fori_loop(0, (rem + L - 1) // L, shift_body, 0)
      return rem, nfl + 1

    return lax.cond(cnt >= FLUSH, do_flush, lambda c: c, (cnt, nfl))

  cnt, nfl = lax.fori_loop(0, n_chunks, chunk_body,
                           (jnp.int32(0), jnp.int32(0)))

  pltpu.sync_copy(idbuf.at[pl.ds(0, FLUSH)],
                  ids_hbm.at[wid, pl.ds(nfl * FLUSH, FLUSH)])
  pltpu.sync_copy(dlbuf.at[pl.ds(0, FLUSH)],
                  dstl_hbm.at[wid, pl.ds(nfl * FLUSH, FLUSH)])

  total = nfl * FLUSH + cnt
  cntv[...] = jnp.zeros((L,), jnp.int32) + total
  pltpu.sync_copy(cntv, cnts_hbm.at[pl.ds(wid * L, L)])


def _sc_prepass(dst):
  mesh = plsc.VectorSubcoreMesh(core_axis_name="c", subcore_axis_name="s")
  f = pl.kernel(
      _prepass_body,
      out_type=(
          jax.ShapeDtypeStruct((NW, CAP), jnp.int32),
          jax.ShapeDtypeStruct((NW, CAP), jnp.int32),
          jax.ShapeDtypeStruct((NW * L,), jnp.int32),
      ),
      mesh=mesh,
      compiler_params=_SC_PARAMS,
      scratch_types=[
          pltpu.VMEM((SCAN_CH,), jnp.int32),
          pltpu.VMEM((BUFW,), jnp.int32),
          pltpu.VMEM((BUFW,), jnp.int32),
          pltpu.VMEM((L,), jnp.int32),
      ],
  )
  return f(dst)


NB_G = EPT // KG
NBP_G = NB_G + 3


def _gather_body(a_hbm, b_hbm, src_hbm, dst_hbm, out_hbm, sidx, didx,
                 ina0, ina1, inb0, inb1, outb0, outb1,
                 sa0, sa1, sb0, sb1, so0, so1):
  wid = _wid()
  e0 = wid * EPT
  cols = [c * L + _iota() for c in range(D // L)]
  ina = (ina0, ina1)
  inb = (inb0, inb1)
  outb = (outb0, outb1)
  sa = (sa0, sa1)
  sb = (sb0, sb1)
  so = (so0, so1)

  pltpu.sync_copy(src_hbm.at[pl.ds(e0, EPT)], sidx)
  pltpu.sync_copy(dst_hbm.at[pl.ds(e0, EPT)], didx)

  def start_in(g, b):
    off = g * KG
    pltpu.async_copy(a_hbm.at[didx.at[pl.ds(off, KG)]], ina[b], sa[b])
    pltpu.async_copy(b_hbm.at[sidx.at[pl.ds(off, KG)]], inb[b], sb[b])

  start_in(0, 0)
  start_in(1, 1)

  def outer(t, _):
    for b in range(2):
      g = t * 2 + b

      @pl.when(g < NB_G)
      def _():
        pltpu.make_async_copy(a_hbm.at[pl.ds(0, KG)], ina[b], sa[b]).wait()
        pltpu.make_async_copy(b_hbm.at[pl.ds(0, KG)], inb[b], sb[b]).wait()

      @pl.when((g >= 2) & (g < NB_G + 2))
      def _():
        pltpu.make_async_copy(outb[b], out_hbm.at[pl.ds(0, KG)], so[b]).wait()

      @pl.when(g < NB_G)
      def _():
        def row_body(i, _):
          for c in range(D // L):
            sl = pl.ds(c * L, L)
            outb[b][i, sl] = jnp.maximum(ina[b][i, sl] + inb[b][i, sl], 0.0)
          return 0
        lax.fori_loop(0, KG, row_body, 0)
        pltpu.async_copy(outb[b], out_hbm.at[pl.ds(e0 + g * KG, KG)], so[b])

      @pl.when(g + 2 < NB_G)
      def _():
        start_in(g + 2, b)
    return 0

  lax.fori_loop(0, (NBP_G + 1) // 2, outer, 0)


def _sc_gather(a, b, src, dst):
  mesh = plsc.VectorSubcoreMesh(core_axis_name="c", subcore_axis_name="s")
  f = pl.kernel(
      _gather_body,
      out_type=jax.ShapeDtypeStruct((N_EDGES, D), jnp.float32),
      mesh=mesh,
      compiler_params=_SC_PARAMS,
      scratch_types=[
          pltpu.VMEM((EPT,), jnp.int32),
          pltpu.VMEM((EPT,), jnp.int32),
          pltpu.VMEM((KG, D), jnp.float32),
          pltpu.VMEM((KG, D), jnp.float32),
          pltpu.VMEM((KG, D), jnp.float32),
          pltpu.VMEM((KG, D), jnp.float32),
          pltpu.VMEM((KG, D), jnp.float32),
          pltpu.VMEM((KG, D), jnp.float32),
          pltpu.SemaphoreType.DMA,
          pltpu.SemaphoreType.DMA,
          pltpu.SemaphoreType.DMA,
          pltpu.SemaphoreType.DMA,
          pltpu.SemaphoreType.DMA,
          pltpu.SemaphoreType.DMA,
      ],
  )
  return f(a, b, src, dst)


def _scatter_body(m_hbm, ids_hbm, dstl_hbm, cnts_hbm, out_hbm, idxb, dlb,
                  rows, cntv, acc, sem, *, init_val, finalize):
  wid = _wid()
  lo = wid * NPT
  cols = [c * L + _iota() for c in range(D // L)]

  def init_body(i, _):
    acc[pl.ds(i * L, L)] = jnp.full((L,), init_val, jnp.float32)
    return 0
  lax.fori_loop(0, (NPT + 1) * D // L, init_body, 0)

  pltpu.sync_copy(cnts_hbm.at[pl.ds(wid * L, L)], cntv)
  cnt = jnp.max(cntv[...])
  nb = (cnt + KS - 1) // KS

  def batch_body(g, _):
    off = g * KS
    pltpu.sync_copy(ids_hbm.at[wid, pl.ds(off, KS)], idxb)
    pltpu.sync_copy(dstl_hbm.at[wid, pl.ds(off, KS)], dlb)
    pltpu.async_copy(m_hbm.at[idxb], rows, sem).wait()

    def edge_body(j, _):
      jv = jnp.zeros((L,), jnp.int32) + j
      base = jnp.max(plsc.load_gather(dlb, [jv])) * D
      for c in range(D // L):
        asl = pl.ds(base + c * L, L)
        cur = acc[asl]
        val = rows[j, pl.ds(c * L, L)]
        acc[asl] = jnp.maximum(cur, val)
      return 0
    lax.fori_loop(0, KS, edge_body, 0)
    return 0

  lax.fori_loop(0, nb, batch_body, 0)

  if finalize:
    def fin_body(i, _):
      sl = pl.ds(i * L, L)
      v = acc[sl]
      acc[sl] = jnp.where(v == jnp.float32(NEG_INIT), 0.0, v)
      return 0
    lax.fori_loop(0, NPT * D // L, fin_body, 0)

  pltpu.sync_copy(acc.at[pl.ds(0, NPT * D)],
                  out_hbm.at[pl.ds(lo * D, NPT * D)])


def _sc_scatter(m, ids, dstl, cnts, init_val, finalize):
  mesh = plsc.VectorSubcoreMesh(core_axis_name="c", subcore_axis_name="s")
  body = functools.partial(_scatter_body, init_val=init_val, finalize=finalize)
  f = pl.kernel(
      body,
      out_type=jax.ShapeDtypeStruct((N_PAD * D,), jnp.float32),
      mesh=mesh,
      compiler_params=_SC_PARAMS,
      scratch_types=[
          pltpu.VMEM((KS,), jnp.int32),
          pltpu.VMEM((KS,), jnp.int32),
          pltpu.VMEM((KS, D), jnp.float32),
          pltpu.VMEM((L,), jnp.int32),
          pltpu.VMEM(((NPT + 1) * D,), jnp.float32),
          pltpu.SemaphoreType.DMA,
      ],
  )
  return f(m, ids, dstl, cnts)


def _ab_body(x_ref, wa_ref, wb_ref, b_ref, oa_ref, ob_ref):
  x = x_ref[...]
  oa_ref[...] = (
      jnp.dot(x, wa_ref[...], preferred_element_type=jnp.float32) + b_ref[...]
  )
  ob_ref[...] = jnp.dot(x, wb_ref[...], preferred_element_type=jnp.float32)


def _tc_ab(x, wa, wb, b1, bm):
  m = x.shape[0]
  return pl.pallas_call(
      _ab_body,
      grid=(m // bm,),
      in_specs=[
          pl.BlockSpec((bm, D), lambda i: (i, 0)),
          pl.BlockSpec((D, D), lambda i: (0, 0)),
          pl.BlockSpec((D, D), lambda i: (0, 0)),
          pl.BlockSpec((1, D), lambda i: (0, 0)),
      ],
      out_specs=[
          pl.BlockSpec((bm, D), lambda i: (i, 0)),
          pl.BlockSpec((bm, D), lambda i: (i, 0)),
      ],
      out_shape=[
          jax.ShapeDtypeStruct((m, D), jnp.float32),
          jax.ShapeDtypeStruct((m, D), jnp.float32),
      ],
  )(x, wa, wb, b1.reshape(1, D))


def _mm_body(x_ref, w_ref, b_ref, o_ref):
  o_ref[...] = (
      jnp.dot(x_ref[...], w_ref[...], preferred_element_type=jnp.float32)
      + b_ref[...]
  )


def _tc_mm(x, w, b, bm):
  m, k = x.shape
  n = w.shape[1]
  return pl.pallas_call(
      _mm_body,
      grid=(m // bm,),
      in_specs=[
          pl.BlockSpec((bm, k), lambda i: (i, 0)),
          pl.BlockSpec((k, n), lambda i: (0, 0)),
          pl.BlockSpec((1, n), lambda i: (0, 0)),
      ],
      out_specs=pl.BlockSpec((bm, n), lambda i: (i, 0)),
      out_shape=jax.ShapeDtypeStruct((m, n), jnp.float32),
  )(x, w, b.reshape(1, n))


def kernel(z, edge_index, l0_W1, l0_b1, l0_W2, l0_b2, l1_W1, l1_b1, l1_W2,
           l1_b2):
  src = edge_index[0].astype(jnp.int32)
  dst = edge_index[1].astype(jnp.int32)

  ids, dstl, cnts = _sc_prepass(dst)

  h = jnp.zeros((N_PAD, D), jnp.float32).at[:N_NODES].set(z)
  for li, (W1, b1, W2, b2) in enumerate(
      [(l0_W1, l0_b1, l0_W2, l0_b2), (l1_W1, l1_b1, l1_W2, l1_b2)]):
    last = li == 1
    a, bmat = _tc_ab(h, W1[:D], W1[D:], b1, bm=1024)
    m1 = _sc_gather(a, bmat, src, dst)
    mm = _tc_mm(m1, W2, b2, bm=2000)
    init = 0.0 if not last else NEG_INIT
    hflat = _sc_scatter(mm, ids, dstl, cnts, init, finalize=last)
    h = hflat.reshape(N_PAD, D)

  return h[:N_NODES]
